# baseline (speedup 1.0000x reference)
import functools

import jax
import jax.numpy as jnp
from jax import lax
from jax.experimental import pallas as pl
from jax.experimental.pallas import tpu as pltpu
from jax.experimental.pallas import tpu_sc as plsc

N_CON = 5000
N_COL = 5000
N = N_CON + N_COL
E = 160000

_HI = jax.lax.Precision.HIGHEST


def _dot(a, b):
    return jnp.dot(a, b, precision=_HI)


CK = 128
E1_CH = 84
E2_CH = 42
EPAD = 16 * E1_CH * CK
ACC1H_R = 2560
NQ = 2500
ACC2_R = 5120


def _tc1_body(x_ref, wn_ref, bn_ref, wc_ref, bc_ref, w1_ref, as_ref,
              g_ref, a_ref):
    i = pl.program_id(0)
    is_con = i < (N_CON // 1000)
    W = jnp.where(is_con, wn_ref[...], wc_ref[...])
    b = jnp.where(is_con, bn_ref[...], bc_ref[...])
    emb = jnp.maximum(_dot(x_ref[...], W) + b, 0.0)
    xh = _dot(emb, w1_ref[...])
    ones = jnp.ones((xh.shape[0], 16), jnp.float32)
    zeros = jnp.zeros((xh.shape[0], 48), jnp.float32)
    g_ref[...] = jnp.stack(
        [jnp.concatenate([xh[:, h * 64:(h + 1) * 64], ones, zeros], axis=1)
         for h in range(4)], axis=0)
    a_ref[...] = jnp.stack([_dot(xh, as_ref[0]), _dot(xh, as_ref[1])], axis=0)


def _tc1(x, wn2, bn, wc2, bc, w1, as1m):
    return pl.pallas_call(
        _tc1_body,
        grid=(N // 1000,),
        in_specs=[
            pl.BlockSpec((1000, 128), lambda i: (i, 0)),
            pl.BlockSpec((128, 64), lambda i: (0, 0)),
            pl.BlockSpec((1, 64), lambda i: (0, 0)),
            pl.BlockSpec((128, 64), lambda i: (0, 0)),
            pl.BlockSpec((1, 64), lambda i: (0, 0)),
            pl.BlockSpec((64, 256), lambda i: (0, 0)),
            pl.BlockSpec((2, 256, 4), lambda i: (0, 0, 0)),
        ],
        out_specs=[
            pl.BlockSpec((4, 1000, 128), lambda i: (0, i, 0)),
            pl.BlockSpec((2, 1000, 4), lambda i: (0, i, 0)),
        ],
        out_shape=[
            jax.ShapeDtypeStruct((4, N, 128), jnp.float32),
            jax.ShapeDtypeStruct((2, N, 4), jnp.float32),
        ],
    )(x, wn2, bn, wc2, bc, w1, as1m)


def _tc2_body(a0_ref, a1_ref, a2_ref, a3_ref, b1_ref, w2_ref, as_ref,
              g_ref, a_ref):
    hs = []
    for r in (a0_ref, a1_ref, a2_ref, a3_ref):
        a = r[0]
        hs.append(a[:, 0:64] / a[:, 64:65])
    emb1 = jnp.maximum(jnp.concatenate(hs, axis=1) + b1_ref[...], 0.0)
    xh2 = _dot(emb1, w2_ref[...])
    ones = jnp.ones((xh2.shape[0], 16), jnp.float32)
    zeros = jnp.zeros((xh2.shape[0], 48), jnp.float32)
    g_ref[...] = jnp.concatenate([xh2, ones, zeros], axis=1)[None]
    a_ref[...] = _dot(xh2, as_ref[...])[None]


def _tc2(acc, b1, w2, as2m):
    return pl.pallas_call(
        _tc2_body,
        grid=(4,),
        in_specs=[pl.BlockSpec((1, ACC1H_R, 128),
                               lambda i: (i, 0, 0))] * 4 + [
            pl.BlockSpec((1, 256), lambda i: (0, 0)),
            pl.BlockSpec((256, 64), lambda i: (0, 0)),
            pl.BlockSpec((64, 2), lambda i: (0, 0)),
        ],
        out_specs=[
            pl.BlockSpec((1, ACC1H_R, 128), lambda i: (i, 0, 0)),
            pl.BlockSpec((1, ACC1H_R, 2), lambda i: (i, 0, 0)),
        ],
        out_shape=[
            jax.ShapeDtypeStruct((4, ACC1H_R, 128), jnp.float32),
            jax.ShapeDtypeStruct((4, ACC1H_R, 2), jnp.float32),
        ],
    )(*acc, b1, w2, as2m)


def _tc3_body(acc0_ref, acc1_ref, b2_ref, wo_ref, bo_ref, out_ref):
    acc = acc0_ref[...] + acc1_ref[...]
    emb2 = jnp.maximum(acc[:, 0:64] / acc[:, 64:65] + b2_ref[...], 0.0)
    out_ref[...] = _dot(emb2, wo_ref[...]) + bo_ref[...]


def _tc3(acc0, acc1, b2, wo, bo):
    return pl.pallas_call(
        _tc3_body,
        grid=(N_COL // 1000,),
        in_specs=[
            pl.BlockSpec((1000, 128), lambda i: (i, 0)),
            pl.BlockSpec((1000, 128), lambda i: (i, 0)),
            pl.BlockSpec((1, 64), lambda i: (0, 0)),
            pl.BlockSpec((64, 64), lambda i: (0, 0)),
            pl.BlockSpec((1, 64), lambda i: (0, 0)),
        ],
        out_specs=pl.BlockSpec((1000, 64), lambda i: (i, 0)),
        out_shape=jax.ShapeDtypeStruct((N_COL, 64), jnp.float32),
    )(acc0, acc1, b2, wo, bo)


@functools.cache
def _mesh():
    return plsc.VectorSubcoreMesh(core_axis_name="c", subcore_axis_name="s",
                                  num_cores=2, num_subcores=16)


_SC_PARAMS = pltpu.CompilerParams(needs_layout_passes=False)


def _zero16():
    return jnp.zeros((16,), jnp.float32)


def _zero_buf(buf):
    @pl.loop(0, CK)
    def _(r):
        @pl.loop(0, 8)
        def _(j):
            buf[r, pl.ds(j * 16, 16)] = _zero16()


def _sc1(g4, a1, srcE, dstE):
    out_t = jax.ShapeDtypeStruct((4, ACC1H_R, 128), jnp.float32)

    @functools.partial(
        pl.kernel,
        out_type=[out_t] * 4,
        mesh=_mesh(),
        compiler_params=_SC_PARAMS,
        scratch_types=[
            pltpu.VMEM((10016 * 4,), jnp.float32),
            pltpu.VMEM((E1_CH, CK), jnp.int32),
            pltpu.VMEM((E1_CH, CK), jnp.int32),
            pltpu.VMEM((E1_CH, CK), jnp.int32),
            pltpu.VMEM((CK,), jnp.float32),
            pltpu.VMEM((CK, 128), jnp.float32),
            pltpu.VMEM((CK, 128), jnp.float32),
            pltpu.VMEM((16, 128), jnp.float32),
            pltpu.VMEM_SHARED((ACC1H_R, 128), jnp.float32),
            pltpu.SemaphoreType.DMA,
            pltpu.SemaphoreType.DMA,
            pltpu.SemaphoreType.DMA,
            pltpu.SemaphoreType.DMA,
        ],
    )
    def k(g0_hbm, g1_hbm, g2_hbm, g3_hbm, a_hbm, src_hbm, dst_hbm,
          o0_hbm, o1_hbm, o2_hbm, o3_hbm,
          av, srcv, dstv, dstmv, exb, rows, rowsB, zbuf, accum,
          semA, semB, semSA, semSB):
        c = lax.axis_index("c")
        s = lax.axis_index("s")
        pltpu.sync_copy(a_hbm.at[c], av)
        pltpu.sync_copy(src_hbm.at[s], srcv)
        pltpu.sync_copy(dst_hbm.at[s], dstv)

        @pl.loop(0, 16)
        def _(r):
            @pl.loop(0, 8)
            def _(j):
                zbuf[r, pl.ds(j * 16, 16)] = _zero16()
        cols = [jnp.full((16,), j, jnp.int32) for j in range(4)]
        four16 = jnp.full((16,), 4, jnp.int32)
        zero16i = jnp.full((16,), 0, jnp.int32)
        nq16 = jnp.full((16,), NQ, jnp.int32)
        rpt = ACC1H_R // 16
        gtabs = ((g0_hbm, g2_hbm), (g1_hbm, g3_hbm))
        otabs = ((o0_hbm, o2_hbm), (o1_hbm, o3_hbm))

        for kk in range(2):
            @pl.loop(0, 4)
            def _(hh):
                hoff16 = jnp.full((16,), hh * NQ, jnp.int32)
                @pl.loop(0, rpt // 16)
                def _(t):
                    pltpu.sync_copy(zbuf,
                                    accum.at[pl.ds(s * rpt + t * 16, 16)])

                plsc.subcore_barrier()

                def gat(buf, sem, g):
                    @pl.when(c == 0)
                    def _():
                        pltpu.async_copy(gtabs[kk][0].at[srcv.at[g]], buf, sem)
                    @pl.when(c == 1)
                    def _():
                        pltpu.async_copy(gtabs[kk][1].at[srcv.at[g]], buf, sem)

                def dwait(buf, sem):
                    pltpu.make_async_copy(
                        gtabs[kk][0].at[pl.ds(0, CK)], buf, sem).wait()

                def work(buf, g):
                    @pl.loop(0, CK // 16)
                    def _(q):
                        src16 = srcv[g, pl.ds(q * 16, 16)]
                        dst16 = dstv[g, pl.ds(q * 16, 16)]
                        a_s = plsc.load_gather(
                            av, [src16 * four16 + cols[2 * kk]])
                        a_d = plsc.load_gather(
                            av, [dst16 * four16 + cols[2 * kk + 1]])
                        e = a_s + a_d
                        e = jnp.maximum(e, 0.2 * e)
                        exb[pl.ds(q * 16, 16)] = jnp.exp(e)
                        loc16 = dst16 - hoff16
                        ok = (loc16 >= zero16i) & (loc16 < nq16)
                        dstmv[g, pl.ds(q * 16, 16)] = jnp.where(
                            ok, loc16, nq16)

                    @pl.loop(0, CK // 16)
                    def _(q):
                        ex16 = exb[pl.ds(q * 16, 16)]
                        for t in range(16):
                            i = q * 16 + t
                            v = jnp.full((16,), ex16[t], jnp.float32)
                            for j in range(5):
                                buf[i, pl.ds(j * 16, 16)] = (
                                    buf[i, pl.ds(j * 16, 16)] * v)

                def scat(buf, sem, g):
                    pltpu.async_copy(buf, accum.at[dstmv.at[g]], sem,
                                     add=True)

                def swait(buf, sem):
                    pltpu.make_async_copy(
                        buf, accum.at[dstmv.at[0]], sem).wait()

                gat(rows, semA, 0)

                @pl.loop(0, E1_CH // 2)
                def _(t):
                    g0 = 2 * t
                    gat(rowsB, semB, g0 + 1)
                    dwait(rows, semA)
                    work(rows, g0)
                    scat(rows, semSA, g0)
                    dwait(rowsB, semB)
                    work(rowsB, g0 + 1)
                    scat(rowsB, semSB, g0 + 1)
                    swait(rows, semSA)
                    @pl.when(t + 1 < E1_CH // 2)
                    def _():
                        gat(rows, semA, g0 + 2)
                    swait(rowsB, semSB)

                plsc.subcore_barrier()

                @pl.loop(0, rpt // 32)
                def _(t):
                    @pl.when(c == 0)
                    def _():
                        pltpu.sync_copy(
                            accum.at[pl.ds(s * rpt + t * 32, 32)],
                            otabs[kk][0].at[hh, pl.ds(s * rpt + t * 32, 32)])
                    @pl.when(c == 1)
                    def _():
                        pltpu.sync_copy(
                            accum.at[pl.ds(s * rpt + t * 32, 32)],
                            otabs[kk][1].at[hh, pl.ds(s * rpt + t * 32, 32)])

                plsc.subcore_barrier()

    return k(g4[0], g4[1], g4[2], g4[3], a1, srcE, dstE)


def _sc2(g2, a2, srcE, dstE):
    out_t = jax.ShapeDtypeStruct((ACC2_R, 128), jnp.float32)

    @functools.partial(
        pl.kernel,
        out_type=[out_t, out_t],
        mesh=_mesh(),
        compiler_params=_SC_PARAMS,
        scratch_types=[
            pltpu.VMEM((4 * ACC1H_R * 2,), jnp.float32),
            pltpu.VMEM((E2_CH, CK), jnp.int32),
            pltpu.VMEM((E2_CH, CK), jnp.int32),
            pltpu.VMEM((E2_CH, CK), jnp.int32),
            pltpu.VMEM((CK,), jnp.float32),
            pltpu.VMEM((CK, 128), jnp.float32),
            pltpu.VMEM((CK, 128), jnp.float32),
            pltpu.VMEM((CK, 128), jnp.float32),
            pltpu.VMEM_SHARED((ACC2_R, 128), jnp.float32),
            pltpu.SemaphoreType.DMA,
            pltpu.SemaphoreType.DMA,
        ],
    )
    def k(g_hbm, a_hbm, src_hbm, dst_hbm, out0_hbm, out1_hbm,
          av, srcv, dstv, dstmv, exb, rows, rowsB, zbuf, accum, semA, semB):
        c = lax.axis_index("c")
        s = lax.axis_index("s")
        w = c * 16 + s
        pltpu.sync_copy(a_hbm, av)
        pltpu.sync_copy(src_hbm.at[w], srcv)
        pltpu.sync_copy(dst_hbm.at[w], dstv)
        _zero_buf(zbuf)

        rpt = ACC2_R // 16
        @pl.loop(0, 2)
        def _(t):
            pltpu.sync_copy(zbuf, accum.at[pl.ds(s * rpt + t * CK, CK)])
        pltpu.sync_copy(zbuf.at[pl.ds(0, rpt - 2 * CK)],
                        accum.at[pl.ds(s * rpt + 2 * CK, rpt - 2 * CK)])

        plsc.subcore_barrier()

        one16 = jnp.full((16,), 1, jnp.int32)
        two16 = jnp.full((16,), 2, jnp.int32)
        ncon16 = jnp.full((16,), N_CON, jnp.int32)
        junk16 = jnp.full((16,), N_COL, jnp.int32)
        pad16 = jnp.full((16,), ACC1H_R - NQ, jnp.int32)
        zero16i = jnp.full((16,), 0, jnp.int32)

        def qrow(n16):
            q = jnp.where(n16 >= jnp.full((16,), NQ, jnp.int32), one16, zero16i)
            q = q + jnp.where(n16 >= jnp.full((16,), 2 * NQ, jnp.int32),
                              one16, zero16i)
            q = q + jnp.where(n16 >= jnp.full((16,), 3 * NQ, jnp.int32),
                              one16, zero16i)
            return n16 + q * pad16

        @pl.loop(0, E2_CH)
        def _(g):
            @pl.loop(0, CK // 16)
            def _(q):
                src16 = srcv[g, pl.ds(q * 16, 16)]
                dst16 = dstv[g, pl.ds(q * 16, 16)]
                srcv[g, pl.ds(q * 16, 16)] = qrow(src16)
                dstmv[g, pl.ds(q * 16, 16)] = jnp.where(
                    dst16 >= ncon16, dst16 - ncon16, junk16)
                dstv[g, pl.ds(q * 16, 16)] = qrow(dst16)

        def gat(buf, sem, g):
            pltpu.async_copy(g_hbm.at[srcv.at[g]], buf, sem)

        def dwait(buf, sem):
            pltpu.make_async_copy(g_hbm.at[pl.ds(0, CK)], buf, sem).wait()

        def work(buf, g):
            @pl.loop(0, CK // 16)
            def _(q):
                srow16 = srcv[g, pl.ds(q * 16, 16)]
                drow16 = dstv[g, pl.ds(q * 16, 16)]
                a_s = plsc.load_gather(av, [srow16 * two16])
                a_d = plsc.load_gather(av, [drow16 * two16 + one16])
                e = a_s + a_d
                e = jnp.maximum(e, 0.2 * e)
                exb[pl.ds(q * 16, 16)] = jnp.exp(e)

            @pl.loop(0, CK // 16)
            def _(q):
                ex16 = exb[pl.ds(q * 16, 16)]
                for t in range(16):
                    i = q * 16 + t
                    v = jnp.full((16,), ex16[t], jnp.float32)
                    for j in range(5):
                        buf[i, pl.ds(j * 16, 16)] = (
                            buf[i, pl.ds(j * 16, 16)] * v)

            pltpu.sync_copy(buf, accum.at[dstmv.at[g]], add=True)

        gat(rows, semA, 0)

        @pl.loop(0, E2_CH // 2)
        def _(t):
            g0 = 2 * t
            gat(rowsB, semB, g0 + 1)
            dwait(rows, semA)
            work(rows, g0)
            @pl.when(t + 1 < E2_CH // 2)
            def _():
                gat(rows, semA, g0 + 2)
            dwait(rowsB, semB)
            work(rowsB, g0 + 1)

        plsc.subcore_barrier()
        @pl.when(c == 0)
        def _():
            pltpu.sync_copy(accum.at[pl.ds(s * rpt, rpt)],
                            out0_hbm.at[pl.ds(s * rpt, rpt)])
        @pl.when(c == 1)
        def _():
            pltpu.sync_copy(accum.at[pl.ds(s * rpt, rpt)],
                            out1_hbm.at[pl.ds(s * rpt, rpt)])

    return k(g2, a2, srcE, dstE)


def kernel(constraints_state, columns_state, edges, Wn, bn, Wc, bc,
           W1, as1, ad1, b1, W2, as2, ad2, b2, Wo, bo):
    f32 = jnp.float32
    x = jnp.concatenate([constraints_state, columns_state], axis=0)
    wn2 = Wn[:128] + Wn[128:]
    wc2 = Wc[:128] + Wc[128:]
    as1m = jnp.zeros((2, 256, 4), f32)
    for c in range(2):
        for k in range(2):
            h = 2 * c + k
            as1m = as1m.at[c, h * 64:(h + 1) * 64, 2 * k].set(as1[h])
            as1m = as1m.at[c, h * 64:(h + 1) * 64, 2 * k + 1].set(ad1[h])
    as2m = jnp.stack([as2[0], ad2[0]], axis=1)

    G1, A1 = _tc1(x, wn2, bn[None, :], wc2, bc[None, :], W1, as1m)
    A1p = jnp.concatenate(
        [A1.reshape(2, N * 4), jnp.zeros((2, 64), f32)], axis=1)

    loop = jnp.arange(N, dtype=edges.dtype)
    padz = jnp.zeros((EPAD - E - N,), edges.dtype)
    src1p = jnp.concatenate([edges[0], loop, padz]).reshape(16, E1_CH, CK)
    dst1p = jnp.concatenate(
        [edges[1], loop, padz + N]).reshape(16, E1_CH, CK)
    acc1 = _sc1(G1, A1p, src1p, dst1p)

    G2, A2 = _tc2(acc1, b1[None, :], W2, as2m)

    src2p = jnp.concatenate([edges[1], loop, padz]).reshape(32, E2_CH, CK)
    dst2p = jnp.concatenate([edges[0], loop, padz]).reshape(32, E2_CH, CK)
    acc2a, acc2b = _sc2(G2.reshape(4 * ACC1H_R, 128),
                        A2.reshape(4 * ACC1H_R * 2), src2p, dst2p)

    return _tc3(acc2a[:N_COL], acc2b[:N_COL], b2[None, :], Wo, bo[None, :])


# speedup vs baseline: 1.0205x; 1.0205x over previous
import functools

import jax
import jax.numpy as jnp
from jax import lax
from jax.experimental import pallas as pl
from jax.experimental.pallas import tpu as pltpu
from jax.experimental.pallas import tpu_sc as plsc

N_CON = 5000
N_COL = 5000
N = N_CON + N_COL
E = 160000

_HI = jax.lax.Precision.HIGHEST


def _dot(a, b):
    return jnp.dot(a, b, precision=_HI)


CK = 128
E1_CH = 84
E2_CH = 42
EPAD = 16 * E1_CH * CK
ACC1H_R = 2560
NQ = 2500
ACC2_R = 5120


def _tc1_body(x_ref, wn_ref, bn_ref, wc_ref, bc_ref, w1_ref, as_ref,
              g_ref, a_ref):
    i = pl.program_id(0)
    is_con = i < (N_CON // 1000)
    W = jnp.where(is_con, wn_ref[...], wc_ref[...])
    b = jnp.where(is_con, bn_ref[...], bc_ref[...])
    emb = jnp.maximum(_dot(x_ref[...], W) + b, 0.0)
    xh = _dot(emb, w1_ref[...])
    ones = jnp.ones((xh.shape[0], 16), jnp.float32)
    zeros = jnp.zeros((xh.shape[0], 48), jnp.float32)
    g_ref[...] = jnp.stack(
        [jnp.concatenate([xh[:, h * 64:(h + 1) * 64], ones, zeros], axis=1)
         for h in range(4)], axis=0)
    a_ref[...] = jnp.stack([_dot(xh, as_ref[0]), _dot(xh, as_ref[1])], axis=0)


def _tc1(x, wn2, bn, wc2, bc, w1, as1m):
    return pl.pallas_call(
        _tc1_body,
        grid=(N // 1000,),
        in_specs=[
            pl.BlockSpec((1000, 128), lambda i: (i, 0)),
            pl.BlockSpec((128, 64), lambda i: (0, 0)),
            pl.BlockSpec((1, 64), lambda i: (0, 0)),
            pl.BlockSpec((128, 64), lambda i: (0, 0)),
            pl.BlockSpec((1, 64), lambda i: (0, 0)),
            pl.BlockSpec((64, 256), lambda i: (0, 0)),
            pl.BlockSpec((2, 256, 4), lambda i: (0, 0, 0)),
        ],
        out_specs=[
            pl.BlockSpec((4, 1000, 128), lambda i: (0, i, 0)),
            pl.BlockSpec((2, 1000, 4), lambda i: (0, i, 0)),
        ],
        out_shape=[
            jax.ShapeDtypeStruct((4, N, 128), jnp.float32),
            jax.ShapeDtypeStruct((2, N, 4), jnp.float32),
        ],
    )(x, wn2, bn, wc2, bc, w1, as1m)


def _tc2_body(a0_ref, a1_ref, a2_ref, a3_ref, b1_ref, w2_ref, as_ref,
              g_ref, a_ref):
    hs = []
    for r in (a0_ref, a1_ref, a2_ref, a3_ref):
        a = r[0]
        hs.append(a[:, 0:64] / a[:, 64:65])
    emb1 = jnp.maximum(jnp.concatenate(hs, axis=1) + b1_ref[...], 0.0)
    xh2 = _dot(emb1, w2_ref[...])
    ones = jnp.ones((xh2.shape[0], 16), jnp.float32)
    zeros = jnp.zeros((xh2.shape[0], 48), jnp.float32)
    g_ref[...] = jnp.concatenate([xh2, ones, zeros], axis=1)[None]
    a_ref[...] = _dot(xh2, as_ref[...])[None]


def _tc2(acc, b1, w2, as2m):
    return pl.pallas_call(
        _tc2_body,
        grid=(4,),
        in_specs=[pl.BlockSpec((1, ACC1H_R, 128),
                               lambda i: (i, 0, 0))] * 4 + [
            pl.BlockSpec((1, 256), lambda i: (0, 0)),
            pl.BlockSpec((256, 64), lambda i: (0, 0)),
            pl.BlockSpec((64, 2), lambda i: (0, 0)),
        ],
        out_specs=[
            pl.BlockSpec((1, ACC1H_R, 128), lambda i: (i, 0, 0)),
            pl.BlockSpec((1, ACC1H_R, 2), lambda i: (i, 0, 0)),
        ],
        out_shape=[
            jax.ShapeDtypeStruct((4, ACC1H_R, 128), jnp.float32),
            jax.ShapeDtypeStruct((4, ACC1H_R, 2), jnp.float32),
        ],
    )(*acc, b1, w2, as2m)


def _tc3_body(acc0_ref, acc1_ref, b2_ref, wo_ref, bo_ref, out_ref):
    acc = acc0_ref[...] + acc1_ref[...]
    emb2 = jnp.maximum(acc[:, 0:64] / acc[:, 64:65] + b2_ref[...], 0.0)
    out_ref[...] = _dot(emb2, wo_ref[...]) + bo_ref[...]


def _tc3(acc0, acc1, b2, wo, bo):
    return pl.pallas_call(
        _tc3_body,
        grid=(N_COL // 1000,),
        in_specs=[
            pl.BlockSpec((1000, 128), lambda i: (i, 0)),
            pl.BlockSpec((1000, 128), lambda i: (i, 0)),
            pl.BlockSpec((1, 64), lambda i: (0, 0)),
            pl.BlockSpec((64, 64), lambda i: (0, 0)),
            pl.BlockSpec((1, 64), lambda i: (0, 0)),
        ],
        out_specs=pl.BlockSpec((1000, 64), lambda i: (i, 0)),
        out_shape=jax.ShapeDtypeStruct((N_COL, 64), jnp.float32),
    )(acc0, acc1, b2, wo, bo)


@functools.cache
def _mesh():
    return plsc.VectorSubcoreMesh(core_axis_name="c", subcore_axis_name="s",
                                  num_cores=2, num_subcores=16)


_SC_PARAMS = pltpu.CompilerParams(needs_layout_passes=False)


def _zero16():
    return jnp.zeros((16,), jnp.float32)


def _zero_buf(buf):
    @pl.loop(0, CK)
    def _(r):
        @pl.loop(0, 8)
        def _(j):
            buf[r, pl.ds(j * 16, 16)] = _zero16()


def _sc1(g4, a1, srcE, dstE):
    out_t = jax.ShapeDtypeStruct((4, ACC1H_R, 128), jnp.float32)

    @functools.partial(
        pl.kernel,
        out_type=[out_t] * 4,
        mesh=_mesh(),
        compiler_params=_SC_PARAMS,
        scratch_types=[
            pltpu.VMEM((10016 * 4,), jnp.float32),
            pltpu.VMEM((E1_CH, CK), jnp.int32),
            pltpu.VMEM((E1_CH, CK), jnp.int32),
            pltpu.VMEM((E1_CH, CK), jnp.int32),
            pltpu.VMEM((CK,), jnp.float32),
            pltpu.VMEM((CK, 128), jnp.float32),
            pltpu.VMEM((CK, 128), jnp.float32),
            pltpu.VMEM((16, 128), jnp.float32),
            pltpu.VMEM_SHARED((ACC1H_R, 128), jnp.float32),
            pltpu.SemaphoreType.DMA,
            pltpu.SemaphoreType.DMA,
        ],
    )
    def k(g0_hbm, g1_hbm, g2_hbm, g3_hbm, a_hbm, src_hbm, dst_hbm,
          o0_hbm, o1_hbm, o2_hbm, o3_hbm,
          av, srcv, dstv, dstmv, exb, rows, rowsB, zbuf, accum, semA, semB):
        c = lax.axis_index("c")
        s = lax.axis_index("s")
        pltpu.sync_copy(a_hbm.at[c], av)
        pltpu.sync_copy(src_hbm.at[s], srcv)
        pltpu.sync_copy(dst_hbm.at[s], dstv)

        @pl.loop(0, 16)
        def _(r):
            @pl.loop(0, 8)
            def _(j):
                zbuf[r, pl.ds(j * 16, 16)] = _zero16()
        cols = [jnp.full((16,), j, jnp.int32) for j in range(4)]
        four16 = jnp.full((16,), 4, jnp.int32)
        zero16i = jnp.full((16,), 0, jnp.int32)
        nq16 = jnp.full((16,), NQ, jnp.int32)
        rpt = ACC1H_R // 16
        gtabs = ((g0_hbm, g2_hbm), (g1_hbm, g3_hbm))
        otabs = ((o0_hbm, o2_hbm), (o1_hbm, o3_hbm))

        for kk in range(2):
            @pl.loop(0, 4)
            def _(hh):
                hoff16 = jnp.full((16,), hh * NQ, jnp.int32)
                @pl.loop(0, rpt // 16)
                def _(t):
                    pltpu.sync_copy(zbuf,
                                    accum.at[pl.ds(s * rpt + t * 16, 16)])

                plsc.subcore_barrier()

                def gat(buf, sem, g):
                    @pl.when(c == 0)
                    def _():
                        pltpu.async_copy(gtabs[kk][0].at[srcv.at[g]], buf, sem)
                    @pl.when(c == 1)
                    def _():
                        pltpu.async_copy(gtabs[kk][1].at[srcv.at[g]], buf, sem)

                def dwait(buf, sem):
                    pltpu.make_async_copy(
                        gtabs[kk][0].at[pl.ds(0, CK)], buf, sem).wait()

                def work(buf, g):
                    @pl.loop(0, CK // 16)
                    def _(q):
                        src16 = srcv[g, pl.ds(q * 16, 16)]
                        dst16 = dstv[g, pl.ds(q * 16, 16)]
                        a_s = plsc.load_gather(
                            av, [src16 * four16 + cols[2 * kk]])
                        a_d = plsc.load_gather(
                            av, [dst16 * four16 + cols[2 * kk + 1]])
                        e = a_s + a_d
                        e = jnp.maximum(e, 0.2 * e)
                        exb[pl.ds(q * 16, 16)] = jnp.exp(e)
                        loc16 = dst16 - hoff16
                        ok = (loc16 >= zero16i) & (loc16 < nq16)
                        dstmv[g, pl.ds(q * 16, 16)] = jnp.where(
                            ok, loc16, nq16)

                    @pl.loop(0, CK // 16)
                    def _(q):
                        ex16 = exb[pl.ds(q * 16, 16)]
                        for t in range(16):
                            i = q * 16 + t
                            v = jnp.full((16,), ex16[t], jnp.float32)
                            for j in range(5):
                                buf[i, pl.ds(j * 16, 16)] = (
                                    buf[i, pl.ds(j * 16, 16)] * v)

                    pltpu.sync_copy(buf, accum.at[dstmv.at[g]], add=True)

                gat(rows, semA, 0)

                @pl.loop(0, E1_CH // 2)
                def _(t):
                    g0 = 2 * t
                    gat(rowsB, semB, g0 + 1)
                    dwait(rows, semA)
                    work(rows, g0)
                    @pl.when(t + 1 < E1_CH // 2)
                    def _():
                        gat(rows, semA, g0 + 2)
                    dwait(rowsB, semB)
                    work(rowsB, g0 + 1)

                plsc.subcore_barrier()

                @pl.loop(0, rpt // 32)
                def _(t):
                    @pl.when(c == 0)
                    def _():
                        pltpu.sync_copy(
                            accum.at[pl.ds(s * rpt + t * 32, 32)],
                            otabs[kk][0].at[hh, pl.ds(s * rpt + t * 32, 32)])
                    @pl.when(c == 1)
                    def _():
                        pltpu.sync_copy(
                            accum.at[pl.ds(s * rpt + t * 32, 32)],
                            otabs[kk][1].at[hh, pl.ds(s * rpt + t * 32, 32)])

                plsc.subcore_barrier()

    return k(g4[0], g4[1], g4[2], g4[3], a1, srcE, dstE)


def _sc2(g2, a2, srcE, dstE):
    out_t = jax.ShapeDtypeStruct((ACC2_R, 128), jnp.float32)

    @functools.partial(
        pl.kernel,
        out_type=[out_t, out_t],
        mesh=_mesh(),
        compiler_params=_SC_PARAMS,
        scratch_types=[
            pltpu.VMEM((4 * ACC1H_R * 2,), jnp.float32),
            pltpu.VMEM((E2_CH, CK), jnp.int32),
            pltpu.VMEM((E2_CH, CK), jnp.int32),
            pltpu.VMEM((E2_CH, CK), jnp.int32),
            pltpu.VMEM((CK,), jnp.float32),
            pltpu.VMEM((CK, 128), jnp.float32),
            pltpu.VMEM((CK, 128), jnp.float32),
            pltpu.VMEM((CK, 128), jnp.float32),
            pltpu.VMEM_SHARED((ACC2_R, 128), jnp.float32),
            pltpu.SemaphoreType.DMA,
            pltpu.SemaphoreType.DMA,
        ],
    )
    def k(g_hbm, a_hbm, src_hbm, dst_hbm, out0_hbm, out1_hbm,
          av, srcv, dstv, dstmv, exb, rows, rowsB, zbuf, accum, semA, semB):
        c = lax.axis_index("c")
        s = lax.axis_index("s")
        w = c * 16 + s
        pltpu.sync_copy(a_hbm, av)
        pltpu.sync_copy(src_hbm.at[w], srcv)
        pltpu.sync_copy(dst_hbm.at[w], dstv)
        _zero_buf(zbuf)

        rpt = ACC2_R // 16
        @pl.loop(0, 2)
        def _(t):
            pltpu.sync_copy(zbuf, accum.at[pl.ds(s * rpt + t * CK, CK)])
        pltpu.sync_copy(zbuf.at[pl.ds(0, rpt - 2 * CK)],
                        accum.at[pl.ds(s * rpt + 2 * CK, rpt - 2 * CK)])

        plsc.subcore_barrier()

        one16 = jnp.full((16,), 1, jnp.int32)
        two16 = jnp.full((16,), 2, jnp.int32)
        ncon16 = jnp.full((16,), N_CON, jnp.int32)
        junk16 = jnp.full((16,), N_COL, jnp.int32)
        pad16 = jnp.full((16,), ACC1H_R - NQ, jnp.int32)
        zero16i = jnp.full((16,), 0, jnp.int32)

        def qrow(n16):
            q = jnp.where(n16 >= jnp.full((16,), NQ, jnp.int32), one16, zero16i)
            q = q + jnp.where(n16 >= jnp.full((16,), 2 * NQ, jnp.int32),
                              one16, zero16i)
            q = q + jnp.where(n16 >= jnp.full((16,), 3 * NQ, jnp.int32),
                              one16, zero16i)
            return n16 + q * pad16

        @pl.loop(0, E2_CH)
        def _(g):
            @pl.loop(0, CK // 16)
            def _(q):
                src16 = srcv[g, pl.ds(q * 16, 16)]
                dst16 = dstv[g, pl.ds(q * 16, 16)]
                srcv[g, pl.ds(q * 16, 16)] = qrow(src16)
                dstmv[g, pl.ds(q * 16, 16)] = jnp.where(
                    dst16 >= ncon16, dst16 - ncon16, junk16)
                dstv[g, pl.ds(q * 16, 16)] = qrow(dst16)

        def gat(buf, sem, g):
            pltpu.async_copy(g_hbm.at[srcv.at[g]], buf, sem)

        def dwait(buf, sem):
            pltpu.make_async_copy(g_hbm.at[pl.ds(0, CK)], buf, sem).wait()

        def work(buf, g):
            @pl.loop(0, CK // 16)
            def _(q):
                srow16 = srcv[g, pl.ds(q * 16, 16)]
                drow16 = dstv[g, pl.ds(q * 16, 16)]
                a_s = plsc.load_gather(av, [srow16 * two16])
                a_d = plsc.load_gather(av, [drow16 * two16 + one16])
                e = a_s + a_d
                e = jnp.maximum(e, 0.2 * e)
                exb[pl.ds(q * 16, 16)] = jnp.exp(e)

            @pl.loop(0, CK // 16)
            def _(q):
                ex16 = exb[pl.ds(q * 16, 16)]
                for t in range(16):
                    i = q * 16 + t
                    v = jnp.full((16,), ex16[t], jnp.float32)
                    for j in range(5):
                        buf[i, pl.ds(j * 16, 16)] = (
                            buf[i, pl.ds(j * 16, 16)] * v)

            pltpu.sync_copy(buf, accum.at[dstmv.at[g]], add=True)

        gat(rows, semA, 0)

        @pl.loop(0, E2_CH // 2)
        def _(t):
            g0 = 2 * t
            gat(rowsB, semB, g0 + 1)
            dwait(rows, semA)
            work(rows, g0)
            @pl.when(t + 1 < E2_CH // 2)
            def _():
                gat(rows, semA, g0 + 2)
            dwait(rowsB, semB)
            work(rowsB, g0 + 1)

        plsc.subcore_barrier()
        @pl.when(c == 0)
        def _():
            pltpu.sync_copy(accum.at[pl.ds(s * rpt, rpt)],
                            out0_hbm.at[pl.ds(s * rpt, rpt)])
        @pl.when(c == 1)
        def _():
            pltpu.sync_copy(accum.at[pl.ds(s * rpt, rpt)],
                            out1_hbm.at[pl.ds(s * rpt, rpt)])

    return k(g2, a2, srcE, dstE)


def kernel(constraints_state, columns_state, edges, Wn, bn, Wc, bc,
           W1, as1, ad1, b1, W2, as2, ad2, b2, Wo, bo):
    f32 = jnp.float32
    x = jnp.concatenate([constraints_state, columns_state], axis=0)
    wn2 = Wn[:128] + Wn[128:]
    wc2 = Wc[:128] + Wc[128:]
    as1m = jnp.zeros((2, 256, 4), f32)
    for c in range(2):
        for k in range(2):
            h = 2 * c + k
            as1m = as1m.at[c, h * 64:(h + 1) * 64, 2 * k].set(as1[h])
            as1m = as1m.at[c, h * 64:(h + 1) * 64, 2 * k + 1].set(ad1[h])
    as2m = jnp.stack([as2[0], ad2[0]], axis=1)

    G1, A1 = _tc1(x, wn2, bn[None, :], wc2, bc[None, :], W1, as1m)
    A1p = jnp.concatenate(
        [A1.reshape(2, N * 4), jnp.zeros((2, 64), f32)], axis=1)

    loop = jnp.arange(N, dtype=edges.dtype)
    padz = jnp.zeros((EPAD - E - N,), edges.dtype)
    src1p = jnp.concatenate([edges[0], loop, padz]).reshape(16, E1_CH, CK)
    dst1p = jnp.concatenate(
        [edges[1], loop, padz + N]).reshape(16, E1_CH, CK)
    acc1 = _sc1(G1, A1p, src1p, dst1p)

    G2, A2 = _tc2(acc1, b1[None, :], W2, as2m)

    src2p = jnp.concatenate([edges[1], loop, padz]).reshape(32, E2_CH, CK)
    dst2p = jnp.concatenate([edges[0], loop, padz]).reshape(32, E2_CH, CK)
    acc2a, acc2b = _sc2(G2.reshape(4 * ACC1H_R, 128),
                        A2.reshape(4 * ACC1H_R * 2), src2p, dst2p)

    return _tc3(acc2a[:N_COL], acc2b[:N_COL], b2[None, :], Wo, bo[None, :])
